# baseline (speedup 1.0000x reference)
import dataclasses
import functools

import jax
import jax.numpy as jnp
from jax import lax
from jax.experimental import pallas as pl
from jax.experimental.pallas import tpu as pltpu
from jax.experimental.pallas import tpu_sc as plsc

N = 10000
DEG = 16
D = 128
NPAD = 10240
E = NPAD * DEG
BIG = 1e9
FLOW_ITERS = 10
DUAL_ITERS = 10
DUAL_STEP = 0.01
DUAL_MOM = 0.9

F32 = jnp.float32


def _sc_params(tc_tiling=None):
    cp = pltpu.CompilerParams()
    fields = pltpu.CompilerParams.__dataclass_fields__
    if "needs_layout_passes" in fields:
        cp = dataclasses.replace(cp, needs_layout_passes=False)
    if tc_tiling is not None and "use_tc_tiling_on_sc" in fields:
        cp = dataclasses.replace(cp, use_tc_tiling_on_sc=tc_tiling)
    return cp


def _k1_body(emb_ref, feat_ref, we_ref, wf_ref, b_ref, out_ref):
    i = pl.program_id(0)
    emb = emb_ref[...]
    nrm = jnp.sqrt(jnp.sum(emb * emb, axis=-1, keepdims=True))
    embn = emb / jnp.maximum(nrm, 1.0)
    e0 = (jnp.dot(embn, we_ref[...], preferred_element_type=F32)
          + jnp.dot(feat_ref[...], wf_ref[...], preferred_element_type=F32)
          + b_ref[...])
    bn = out_ref.shape[0]
    rows = i * bn + lax.broadcasted_iota(jnp.int32, e0.shape, 0)
    out_ref[...] = jnp.where(rows < N, e0, 0.0)


def _k1(embp, featp, we, wf, b):
    bn = 1024
    return pl.pallas_call(
        _k1_body,
        grid=(NPAD // bn,),
        in_specs=[
            pl.BlockSpec((bn, D), lambda i: (i, 0)),
            pl.BlockSpec((bn, D), lambda i: (i, 0)),
            pl.BlockSpec((D, D), lambda i: (0, 0)),
            pl.BlockSpec((D, D), lambda i: (0, 0)),
            pl.BlockSpec((1, D), lambda i: (0, 0)),
        ],
        out_specs=pl.BlockSpec((bn, D), lambda i: (i, 0)),
        out_shape=jax.ShapeDtypeStruct((NPAD, D), F32),
    )(embp, featp, we, wf, b)


def _k2_body(h_ref, e0_ref, wt_ref, wb_ref, b_ref, out_ref, row0=0):
    i = pl.program_id(0)
    nr = h_ref[:, 0, :].astype(F32)
    for k in range(1, DEG):
        nr = nr + h_ref[:, k, :].astype(F32)
    bn = out_ref.shape[0]
    rows = row0 + i * bn + lax.broadcasted_iota(jnp.int32, nr.shape, 0)
    nr = jnp.where(rows < N, nr, 0.0)
    p = (jnp.dot(nr, wt_ref[...], preferred_element_type=F32)
         + jnp.dot(e0_ref[...], wb_ref[...], preferred_element_type=F32)
         + b_ref[...])
    out_ref[...] = jnp.tanh(p)


def _k2(h, e0p, wt, wb, b, row0=0):
    bn = 512
    nrows = h.shape[0]
    return pl.pallas_call(
        functools.partial(_k2_body, row0=row0),
        grid=(nrows // bn,),
        in_specs=[
            pl.BlockSpec((bn, DEG, D), lambda i: (i, 0, 0)),
            pl.BlockSpec((bn, D), lambda i: (i, 0)),
            pl.BlockSpec((D, D), lambda i: (0, 0)),
            pl.BlockSpec((D, D), lambda i: (0, 0)),
            pl.BlockSpec((1, D), lambda i: (0, 0)),
        ],
        out_specs=pl.BlockSpec((bn, D), lambda i: (i, 0)),
        out_shape=jax.ShapeDtypeStruct((nrows, D), F32),
    )(h, e0p, wt, wb, b)


def _k3_body(gt_ref, h_ref, adj_ref, ga_ref,
             wz_ref, uz_ref, bz_ref, wr_ref, ur_ref, br_ref,
             wh_ref, uh_ref, bh_ref, out_ref):
    bn = gt_ref.shape[0]
    gt = gt_ref[...].astype(F32)
    scores = jnp.sum(gt * ga_ref[...].reshape(1, 1, D), axis=-1)
    mask = jnp.where(adj_ref[...] == N, 1.0, 0.0).astype(F32)
    sc = scores - BIG * mask
    m = jnp.max(sc, axis=-1, keepdims=True)
    ex = jnp.exp(sc - m)
    attn = ex / jnp.sum(ex, axis=-1, keepdims=True)
    BF = jnp.bfloat16
    x = (attn[..., None] * gt).reshape(bn * DEG, D).astype(BF)
    hf = h_ref[...].astype(F32).reshape(bn * DEG, D)
    h16 = hf.astype(BF)
    z = jax.nn.sigmoid(jnp.dot(x, wz_ref[...].astype(BF), preferred_element_type=F32)
                       + jnp.dot(h16, uz_ref[...].astype(BF), preferred_element_type=F32)
                       + bz_ref[...])
    r = jax.nn.sigmoid(jnp.dot(x, wr_ref[...].astype(BF), preferred_element_type=F32)
                       + jnp.dot(h16, ur_ref[...].astype(BF), preferred_element_type=F32)
                       + br_ref[...])
    ht = jnp.tanh(jnp.dot(x, wh_ref[...].astype(BF), preferred_element_type=F32)
                  + jnp.dot((r * hf).astype(BF), uh_ref[...].astype(BF),
                            preferred_element_type=F32)
                  + bh_ref[...])
    out_ref[...] = (z * hf + (1.0 - z) * ht).reshape(bn, DEG, D)


def _k3(gt, h, adjp, ga, wz, uz, bz, wr, ur, br, wh, uh, bh):
    bn = 512
    nrows = h.shape[0]
    wspec = pl.BlockSpec((D, D), lambda i: (0, 0))
    bspec = pl.BlockSpec((1, D), lambda i: (0, 0))
    return pl.pallas_call(
        _k3_body,
        grid=(nrows // bn,),
        in_specs=[
            pl.BlockSpec((bn, DEG, D), lambda i: (i, 0, 0)),
            pl.BlockSpec((bn, DEG, D), lambda i: (i, 0, 0)),
            pl.BlockSpec((bn, DEG), lambda i: (i, 0)),
            bspec, wspec, wspec, bspec, wspec, wspec, bspec, wspec, wspec, bspec,
        ],
        out_specs=pl.BlockSpec((bn, DEG, D), lambda i: (i, 0, 0)),
        out_shape=jax.ShapeDtypeStruct((nrows, DEG, D), F32),
    )(gt, h, adjp, ga, wz, uz, bz, wr, ur, br, wh, uh, bh)


def _k4_body(h_ref, w1_ref, b1_ref, w2_ref, b2_ref,
             dw1_ref, db1_ref, dw2_ref, db2_ref, w_out, dv_out, row0=0):
    i = pl.program_id(0)
    bn = h_ref.shape[0]
    hf = h_ref[...].reshape(bn * DEG, D)
    t1 = jnp.tanh(jnp.dot(hf.astype(jnp.bfloat16),
                          w1_ref[...].astype(jnp.bfloat16),
                          preferred_element_type=F32)
                  + b1_ref[...])
    w = jnp.sum(t1 * w2_ref[...], axis=-1) + b2_ref[0, 0]
    w_out[...] = w.reshape(bn, DEG)
    ns = h_ref[:, 0, :]
    for k in range(1, DEG):
        ns = ns + h_ref[:, k, :]
    d1 = jnp.tanh(jnp.dot(ns, dw1_ref[...], preferred_element_type=F32)
                  + db1_ref[...])
    dv = jnp.sum(d1 * dw2_ref[...], axis=-1) + db2_ref[0, 0]
    rows = row0 + i * bn + lax.broadcasted_iota(jnp.int32, dv.shape, 0)
    dv = jnp.where(rows < N, dv, 0.0)
    dv_out[...] = jnp.broadcast_to(dv[:, None], (bn, DEG))


def _k4(h, w1, b1, w2, b2, dw1, db1, dw2, db2, row0=0):
    bn = 512
    nrows = h.shape[0]
    wspec = pl.BlockSpec((D, D), lambda i: (0, 0))
    bspec = pl.BlockSpec((1, D), lambda i: (0, 0))
    sspec = pl.BlockSpec((1, 1), lambda i: (0, 0))
    return pl.pallas_call(
        functools.partial(_k4_body, row0=row0),
        grid=(nrows // bn,),
        in_specs=[
            pl.BlockSpec((bn, DEG, D), lambda i: (i, 0, 0)),
            wspec, bspec, bspec, sspec, wspec, bspec, bspec, sspec,
        ],
        out_specs=[
            pl.BlockSpec((bn, DEG), lambda i: (i, 0)),
            pl.BlockSpec((bn, DEG), lambda i: (i, 0)),
        ],
        out_shape=[
            jax.ShapeDtypeStruct((nrows, DEG), F32),
            jax.ShapeDtypeStruct((nrows, DEG), F32),
        ],
    )(h, w1, b1, w2, b2, dw1, db1, dw2, db2)


def _row_gather(table, idx):
    mesh = plsc.VectorSubcoreMesh(core_axis_name="c", subcore_axis_name="s")
    win = 128
    w = table.shape[1]

    @functools.partial(
        pl.kernel,
        out_type=jax.ShapeDtypeStruct((E, w), table.dtype),
        mesh=mesh,
        compiler_params=_sc_params(tc_tiling=False if w < 128 else None),
    )
    def gk(tab_hbm, idx_hbm, out_hbm):
        def body(i_vmem, o_vmem):
            pltpu.sync_copy(tab_hbm.at[i_vmem.at[0]], o_vmem)

        pltpu.emit_pipeline(
            body,
            grid=(E // win,),
            in_specs=[pl.BlockSpec((1, win), index_map=lambda i: (0, i))],
            out_specs=[pl.BlockSpec((win, w), index_map=lambda i: (i, 0))],
            core_axis_name=("c", "s"),
            dimension_semantics=(pltpu.PARALLEL,),
        )(idx_hbm, out_hbm)

    return gk(table, idx.reshape(1, E))


def _row_gather2(table, idx):
    mesh = plsc.VectorSubcoreMesh(core_axis_name="c", subcore_axis_name="s")
    win = 64
    w = table.shape[1]
    ne = idx.shape[0]
    e_per = ne // 32
    nw = e_per // win
    assert e_per % win == 0
    nbuf = next(d for d in (8, 5, 4, 2, 1) if nw % d == 0)
    ngrp = nw // nbuf

    @functools.partial(
        pl.kernel,
        out_type=jax.ShapeDtypeStruct((ne, w), table.dtype),
        mesh=mesh,
        compiler_params=_sc_params(),
        scratch_types=[
            pltpu.VMEM((e_per,), jnp.int32),
            pltpu.VMEM((nbuf, win, w), table.dtype),
            pltpu.SemaphoreType.DMA,
            pltpu.SemaphoreType.DMA,
        ],
    )
    def gk(tab_hbm, idx_hbm, out_hbm, idx_v, rbuf, gsem, ssem):
        wid = lax.axis_index("s") * 2 + lax.axis_index("c")
        base = wid * e_per
        pltpu.sync_copy(idx_hbm.at[pl.ds(base, e_per)], idx_v)

        def g_copy(wi, b):
            return pltpu.make_async_copy(
                tab_hbm.at[idx_v.at[pl.ds(wi * win, win)]], rbuf.at[b], gsem)

        def s_copy(wi, b):
            return pltpu.make_async_copy(
                rbuf.at[b], out_hbm.at[pl.ds(base + wi * win, win)], ssem)

        for b in range(nbuf):
            g_copy(b, b).start()

        @pl.loop(0, ngrp)
        def _(g):
            for b in range(nbuf):
                wi = g * nbuf + b
                g_copy(wi, b).wait()
                s_copy(wi, b).start()
            for b in range(nbuf):
                wn = (g + 1) * nbuf + b

                @pl.when(wn < nw)
                def _():
                    s_copy(g * nbuf + b, b).wait()
                    g_copy(wn, b).start()

        for b in range(nbuf):
            s_copy((ngrp - 1) * nbuf + b, b).wait()

    return gk(table, idx)


_CHUNK_N = NPAD // 16
_CHUNK_E = _CHUNK_N * DEG
_WIN = 128
_NWIN = _CHUNK_E // _WIN


def _solver(w_flat, dv_flat, ii, rv, adjf, invf, iit, dem_node):
    mesh = plsc.VectorSubcoreMesh(core_axis_name="c", subcore_axis_name="s")
    grp = _CHUNK_N // 16

    @functools.partial(
        pl.kernel,
        out_type=(
            jax.ShapeDtypeStruct((E,), F32),
            jax.ShapeDtypeStruct((16, 16), F32),
            jax.ShapeDtypeStruct((E,), F32),
            jax.ShapeDtypeStruct((E,), F32),
            jax.ShapeDtypeStruct((NPAD,), F32),
        ),
        mesh=mesh,
        compiler_params=_sc_params(),
        scratch_types=[
            pltpu.VMEM((_CHUNK_E,), jnp.int32),
            pltpu.VMEM((_CHUNK_E,), jnp.int32),
            pltpu.VMEM((_CHUNK_E,), jnp.int32),
            pltpu.VMEM((_CHUNK_E,), jnp.int32),
            pltpu.VMEM((_CHUNK_E,), jnp.int32),
            pltpu.VMEM((_CHUNK_E,), F32),
            pltpu.VMEM((_CHUNK_E,), F32),
            pltpu.VMEM((_CHUNK_E,), F32),
            pltpu.VMEM((NPAD,), F32),
            pltpu.VMEM((_CHUNK_N,), F32),
            pltpu.VMEM((_CHUNK_N,), F32),
            pltpu.VMEM((16,), F32),
            pltpu.SemaphoreType.DMA,
        ],
    )
    def sk(w_hbm, dv_hbm, ii_hbm, rv_hbm, adj_hbm, inv_hbm, iit_hbm, dem_hbm,
           dtr_hbm, fc_hbm, wt_hbm, nrm_hbm, s_hbm,
           ii_v, rv_v, adj_v, inv_v, ti_v, gath_v, nrm_v, nin_v,
           tab_v, sv_v, dem_v, acc_v, sem):
        cid = lax.axis_index("c")
        sid = lax.axis_index("s")

        @pl.when(cid == 0)
        def _():
            be = sid * _CHUNK_E
            bn0 = sid * _CHUNK_N
            pltpu.sync_copy(ii_hbm.at[pl.ds(be, _CHUNK_E)], ii_v)
            pltpu.sync_copy(rv_hbm.at[pl.ds(be, _CHUNK_E)], rv_v)
            pltpu.sync_copy(adj_hbm.at[pl.ds(be, _CHUNK_E)], adj_v)
            pltpu.sync_copy(inv_hbm.at[pl.ds(be, _CHUNK_E)], inv_v)
            pltpu.sync_copy(iit_hbm.at[pl.ds(be, _CHUNK_E)], ti_v)
            pltpu.sync_copy(dem_hbm.at[pl.ds(bn0, _CHUNK_N)], dem_v)

            def gather_all(tab, idx_ref, out_ref):
                @pl.loop(0, _NWIN)
                def _start(wi):
                    pltpu.make_async_copy(
                        tab.at[idx_ref.at[pl.ds(wi * _WIN, _WIN)]],
                        out_ref.at[pl.ds(wi * _WIN, _WIN)], sem).start()

                @pl.loop(0, _NWIN)
                def _drain(wi):
                    pltpu.make_async_copy(
                        tab.at[idx_ref.at[pl.ds(wi * _WIN, _WIN)]],
                        out_ref.at[pl.ds(wi * _WIN, _WIN)], sem).wait()

            gather_all(w_hbm, ii_v, gath_v)

            @pl.loop(0, _CHUNK_N)
            def _watt(n):
                e0 = n * DEG
                inc = gath_v[pl.ds(e0, DEG)]
                im = jnp.where(inv_v[pl.ds(e0, DEG)] == N, 1.0, 0.0).astype(F32)
                s = inc - BIG * im
                mx = jnp.max(s)
                ex = jnp.exp(s - mx)
                att = ex / jnp.sum(ex)
                nrm_v[pl.ds(e0, DEG)] = att * inc

            pltpu.sync_copy(nrm_v, wt_hbm.at[pl.ds(be, _CHUNK_E)])
            plsc.subcore_barrier()

            gather_all(wt_hbm, rv_v, gath_v)

            @pl.loop(0, _CHUNK_N)
            def _nrm(n):
                e0 = n * DEG
                nw = gath_v[pl.ds(e0, DEG)]
                mk = jnp.where(adj_v[pl.ds(e0, DEG)] == N, 1.0, 0.0).astype(F32)
                s = nw - BIG * mk
                mx = jnp.max(s)
                ex = jnp.exp(s - mx)
                nrm_v[pl.ds(e0, DEG)] = ex / jnp.sum(ex)

            pltpu.sync_copy(nrm_v, nrm_hbm.at[pl.ds(be, _CHUNK_E)])
            pltpu.sync_copy(dem_v, s_hbm.at[pl.ds(bn0, _CHUNK_N)])

            pltpu.sync_copy(dv_hbm, tab_v)

            @pl.loop(0, _CHUNK_N)
            def _dtr(n):
                e0 = n * DEG
                gath_v[pl.ds(e0, DEG)] = plsc.load_gather(
                    tab_v, [adj_v[pl.ds(e0, DEG)]])

            pltpu.sync_copy(gath_v, dtr_hbm.at[pl.ds(be, _CHUNK_E)])
            plsc.subcore_barrier()

            gather_all(nrm_hbm, ti_v, nin_v)

            for _ in range(FLOW_ITERS):
                pltpu.sync_copy(s_hbm, tab_v)

                @pl.loop(0, grp)
                def _grp(g):
                    acc = jnp.zeros((DEG,), F32)
                    for k in range(DEG):
                        off = g * 256 + k * 16
                        jn = lax.shift_right_logical(ti_v[pl.ds(off, 16)], 4)
                        acc = acc + nin_v[pl.ds(off, 16)] * plsc.load_gather(
                            tab_v, [jn])
                    val = jnp.where(bn0 + g * 16 < N, 1.0, 0.0).astype(F32)
                    sv_v[pl.ds(g * 16, 16)] = (
                        (acc + dem_v[pl.ds(g * 16, 16)]) * val)

                pltpu.sync_copy(sv_v, s_hbm.at[pl.ds(bn0, _CHUNK_N)])
                plsc.subcore_barrier()

            acc_v[...] = jnp.zeros((16,), F32)

            @pl.loop(0, _CHUNK_N)
            def _cost(n):
                sval = plsc.load_gather(sv_v, [jnp.full((16,), n, jnp.int32)])
                f = nrm_v[pl.ds(n * DEG, DEG)] * sval
                acc_v[...] = acc_v[...] + f * f

            pltpu.sync_copy(acc_v, fc_hbm.at[sid])

    return sk(w_flat, dv_flat, ii, rv, adjf, invf, iit, dem_node)


def _k6_body(dtr_ref, dv_ref, adj_ref, dem_ref, fc_ref, out_ref):
    i = pl.program_id(0)
    bn = dtr_ref.shape[0]
    mask = jnp.where(adj_ref[...] == N, 1.0, 0.0).astype(F32)
    dv = dv_ref[...]
    diff = dtr_ref[...] - mask * dv
    df = jnp.zeros_like(diff)
    vel = jnp.zeros_like(diff)
    for _ in range(DUAL_ITERS):
        g = 2.0 * df + diff
        vel = DUAL_MOM * vel - DUAL_STEP * g
        df = jnp.maximum(df + vel, 0.0)
    dflows = df * (1.0 - mask)
    rows = i * bn + lax.broadcasted_iota(jnp.int32, diff.shape, 0)
    valid = jnp.where(rows < N, 1.0, 0.0).astype(F32)
    dfc = (dflows * dflows + diff * dflows) * valid
    lane0 = jnp.where(lax.broadcasted_iota(jnp.int32, diff.shape, 1) == 0,
                      1.0, 0.0).astype(F32)
    dual_demand = jnp.sum(dv * dem_ref[...] * lane0)
    contrib = dual_demand - jnp.sum(dfc)

    @pl.when(i == 0)
    def _():
        out_ref[...] = jnp.reshape(contrib + jnp.sum(fc_ref[...]), (1, 1))

    @pl.when(i > 0)
    def _():
        out_ref[...] = out_ref[...] + jnp.reshape(contrib, (1, 1))


def _k6(dtr16, dv16, adjp, dem16, fc):
    bn = 512
    return pl.pallas_call(
        _k6_body,
        grid=(NPAD // bn,),
        in_specs=[
            pl.BlockSpec((bn, DEG), lambda i: (i, 0)),
            pl.BlockSpec((bn, DEG), lambda i: (i, 0)),
            pl.BlockSpec((bn, DEG), lambda i: (i, 0)),
            pl.BlockSpec((bn, DEG), lambda i: (i, 0)),
            pl.BlockSpec((16, 16), lambda i: (0, 0)),
        ],
        out_specs=pl.BlockSpec((1, 1), lambda i: (0, 0)),
        out_shape=jax.ShapeDtypeStruct((1, 1), F32),
    )(dtr16, dv16, adjp, dem16, fc)


def kernel(demands, node_features, adj_lst, inv_adj_lst, in_indices,
           rev_indices, num_nodes, node_embedding_var, enc_W, enc_b,
           gat_W, gat_b, gat_a,
           gru_Wz, gru_Uz, gru_bz, gru_Wr, gru_Ur, gru_br,
           gru_Wh, gru_Uh, gru_bh,
           dec_W1, dec_b1, dec_W2, dec_b2,
           dual_W1, dual_b1, dual_W2, dual_b2):
    embp = jnp.pad(node_embedding_var, ((0, NPAD - N), (0, 0)))
    featp = jnp.pad(node_features[0], ((0, NPAD - N), (0, 0)))
    adjp = jnp.pad(adj_lst[0], ((0, NPAD - N), (0, 0)), constant_values=N)
    invp = jnp.pad(inv_adj_lst[0], ((0, NPAD - N), (0, 0)), constant_values=N)
    ii = jnp.pad(in_indices[0].reshape(-1), (0, E - N * DEG))
    rv = jnp.pad(rev_indices[0].reshape(-1), (0, E - N * DEG))
    adjf = adjp.reshape(-1)
    invf = invp.reshape(-1)
    dem = jnp.pad(demands[0, :, 0], (0, NPAD - N))
    b2 = lambda v: v.reshape(1, -1)

    e0p = _k1(embp, featp, enc_W[:D], enc_W[D:], b2(enc_b))

    P = 4
    NH = NPAD // P
    EH = E // P
    adjf_p = [adjf[i * EH:(i + 1) * EH] for i in range(P)]
    adjp_p = [adjp[i * NH:(i + 1) * NH] for i in range(P)]
    e0_p = [e0p[i * NH:(i + 1) * NH] for i in range(P)]
    hs = [_row_gather2(e0p, adjf_p[i]).reshape(NH, DEG, D) for i in range(P)]
    for _ in range(2):
        ts = [_k2(hs[i], e0_p[i], gat_W[:D], gat_W[D:], b2(gat_b),
                  row0=i * NH) for i in range(P)]
        t = jnp.concatenate(ts, axis=0)
        gts = [_row_gather2(t, adjf_p[i]).reshape(NH, DEG, D)
               for i in range(P)]
        hs = [_k3(gts[i], hs[i], adjp_p[i], b2(gat_a),
                  gru_Wz, gru_Uz, b2(gru_bz),
                  gru_Wr, gru_Ur, b2(gru_br),
                  gru_Wh, gru_Uh, b2(gru_bh)) for i in range(P)]

    heads = [_k4(hs[i], dec_W1, b2(dec_b1), b2(dec_W2[:, 0]),
                 dec_b2.reshape(1, 1),
                 dual_W1, b2(dual_b1), b2(dual_W2[:, 0]),
                 dual_b2.reshape(1, 1), row0=i * NH) for i in range(P)]
    w16 = jnp.concatenate([hd[0] for hd in heads], axis=0)
    dv16 = jnp.concatenate([hd[1] for hd in heads], axis=0)

    dem16 = jnp.broadcast_to(dem[:, None], (NPAD, DEG))
    iit = ii.reshape(NPAD // 16, 16, 16).swapaxes(1, 2).reshape(-1)
    dtr, fc, _, _, _ = _solver(w16.reshape(-1), dv16[:, 0], ii, rv,
                               adjf, invf, iit, dem)
    loss = _k6(dtr.reshape(NPAD, DEG), dv16, adjp, dem16, fc)
    return loss.reshape(1)


# speedup vs baseline: 1.0880x; 1.0880x over previous
import dataclasses
import functools

import jax
import jax.numpy as jnp
from jax import lax
from jax.experimental import pallas as pl
from jax.experimental.pallas import tpu as pltpu
from jax.experimental.pallas import tpu_sc as plsc

N = 10000
DEG = 16
D = 128
NPAD = 10240
E = NPAD * DEG
BIG = 1e9
FLOW_ITERS = 10
DUAL_ITERS = 10
DUAL_STEP = 0.01
DUAL_MOM = 0.9

F32 = jnp.float32


def _sc_params(tc_tiling=None):
    cp = pltpu.CompilerParams()
    fields = pltpu.CompilerParams.__dataclass_fields__
    if "needs_layout_passes" in fields:
        cp = dataclasses.replace(cp, needs_layout_passes=False)
    if tc_tiling is not None and "use_tc_tiling_on_sc" in fields:
        cp = dataclasses.replace(cp, use_tc_tiling_on_sc=tc_tiling)
    return cp


def _k1_body(emb_ref, feat_ref, we_ref, wf_ref, b_ref, out_ref):
    i = pl.program_id(0)
    emb = emb_ref[...]
    nrm = jnp.sqrt(jnp.sum(emb * emb, axis=-1, keepdims=True))
    embn = emb / jnp.maximum(nrm, 1.0)
    e0 = (jnp.dot(embn, we_ref[...], preferred_element_type=F32)
          + jnp.dot(feat_ref[...], wf_ref[...], preferred_element_type=F32)
          + b_ref[...])
    bn = out_ref.shape[0]
    rows = i * bn + lax.broadcasted_iota(jnp.int32, e0.shape, 0)
    out_ref[...] = jnp.where(rows < N, e0, 0.0)


def _k1(embp, featp, we, wf, b):
    bn = 1024
    return pl.pallas_call(
        _k1_body,
        grid=(NPAD // bn,),
        in_specs=[
            pl.BlockSpec((bn, D), lambda i: (i, 0)),
            pl.BlockSpec((bn, D), lambda i: (i, 0)),
            pl.BlockSpec((D, D), lambda i: (0, 0)),
            pl.BlockSpec((D, D), lambda i: (0, 0)),
            pl.BlockSpec((1, D), lambda i: (0, 0)),
        ],
        out_specs=pl.BlockSpec((bn, D), lambda i: (i, 0)),
        out_shape=jax.ShapeDtypeStruct((NPAD, D), F32),
    )(embp, featp, we, wf, b)


def _k2_body(h_ref, e0_ref, wt_ref, wb_ref, b_ref, out_ref, row0=0):
    i = pl.program_id(0)
    nr = h_ref[:, 0, :].astype(F32)
    for k in range(1, DEG):
        nr = nr + h_ref[:, k, :].astype(F32)
    bn = out_ref.shape[0]
    rows = row0 + i * bn + lax.broadcasted_iota(jnp.int32, nr.shape, 0)
    nr = jnp.where(rows < N, nr, 0.0)
    p = (jnp.dot(nr, wt_ref[...], preferred_element_type=F32)
         + jnp.dot(e0_ref[...], wb_ref[...], preferred_element_type=F32)
         + b_ref[...])
    out_ref[...] = jnp.tanh(p)


def _k2(h, e0p, wt, wb, b, row0=0):
    bn = 512
    nrows = h.shape[0]
    return pl.pallas_call(
        functools.partial(_k2_body, row0=row0),
        grid=(nrows // bn,),
        in_specs=[
            pl.BlockSpec((bn, DEG, D), lambda i: (i, 0, 0)),
            pl.BlockSpec((bn, D), lambda i: (i, 0)),
            pl.BlockSpec((D, D), lambda i: (0, 0)),
            pl.BlockSpec((D, D), lambda i: (0, 0)),
            pl.BlockSpec((1, D), lambda i: (0, 0)),
        ],
        out_specs=pl.BlockSpec((bn, D), lambda i: (i, 0)),
        out_shape=jax.ShapeDtypeStruct((nrows, D), F32),
    )(h, e0p, wt, wb, b)


def _k3_body(gt_ref, h_ref, adj_ref, ga_ref,
             wz_ref, uz_ref, bz_ref, wr_ref, ur_ref, br_ref,
             wh_ref, uh_ref, bh_ref, out_ref):
    bn = gt_ref.shape[0]
    gt = gt_ref[...].astype(F32)
    scores = jnp.sum(gt * ga_ref[...].reshape(1, 1, D), axis=-1)
    mask = jnp.where(adj_ref[...] == N, 1.0, 0.0).astype(F32)
    sc = scores - BIG * mask
    m = jnp.max(sc, axis=-1, keepdims=True)
    ex = jnp.exp(sc - m)
    attn = ex / jnp.sum(ex, axis=-1, keepdims=True)
    BF = jnp.bfloat16
    x = (attn[..., None] * gt).reshape(bn * DEG, D).astype(BF)
    hf = h_ref[...].astype(F32).reshape(bn * DEG, D)
    h16 = hf.astype(BF)
    z = jax.nn.sigmoid(jnp.dot(x, wz_ref[...].astype(BF), preferred_element_type=F32)
                       + jnp.dot(h16, uz_ref[...].astype(BF), preferred_element_type=F32)
                       + bz_ref[...])
    r = jax.nn.sigmoid(jnp.dot(x, wr_ref[...].astype(BF), preferred_element_type=F32)
                       + jnp.dot(h16, ur_ref[...].astype(BF), preferred_element_type=F32)
                       + br_ref[...])
    ht = jnp.tanh(jnp.dot(x, wh_ref[...].astype(BF), preferred_element_type=F32)
                  + jnp.dot((r * hf).astype(BF), uh_ref[...].astype(BF),
                            preferred_element_type=F32)
                  + bh_ref[...])
    out_ref[...] = (z * hf + (1.0 - z) * ht).reshape(bn, DEG, D)


def _k3(gt, h, adjp, ga, wz, uz, bz, wr, ur, br, wh, uh, bh):
    bn = 512
    nrows = h.shape[0]
    wspec = pl.BlockSpec((D, D), lambda i: (0, 0))
    bspec = pl.BlockSpec((1, D), lambda i: (0, 0))
    return pl.pallas_call(
        _k3_body,
        grid=(nrows // bn,),
        in_specs=[
            pl.BlockSpec((bn, DEG, D), lambda i: (i, 0, 0)),
            pl.BlockSpec((bn, DEG, D), lambda i: (i, 0, 0)),
            pl.BlockSpec((bn, DEG), lambda i: (i, 0)),
            bspec, wspec, wspec, bspec, wspec, wspec, bspec, wspec, wspec, bspec,
        ],
        out_specs=pl.BlockSpec((bn, DEG, D), lambda i: (i, 0, 0)),
        out_shape=jax.ShapeDtypeStruct((nrows, DEG, D), F32),
    )(gt, h, adjp, ga, wz, uz, bz, wr, ur, br, wh, uh, bh)


def _k4_body(h_ref, w1_ref, b1_ref, w2_ref, b2_ref,
             dw1_ref, db1_ref, dw2_ref, db2_ref, w_out, dv_out, row0=0):
    i = pl.program_id(0)
    bn = h_ref.shape[0]
    hf = h_ref[...].reshape(bn * DEG, D)
    t1 = jnp.tanh(jnp.dot(hf.astype(jnp.bfloat16),
                          w1_ref[...].astype(jnp.bfloat16),
                          preferred_element_type=F32)
                  + b1_ref[...])
    w = jnp.sum(t1 * w2_ref[...], axis=-1) + b2_ref[0, 0]
    w_out[...] = w.reshape(bn, DEG)
    ns = h_ref[:, 0, :]
    for k in range(1, DEG):
        ns = ns + h_ref[:, k, :]
    d1 = jnp.tanh(jnp.dot(ns, dw1_ref[...], preferred_element_type=F32)
                  + db1_ref[...])
    dv = jnp.sum(d1 * dw2_ref[...], axis=-1) + db2_ref[0, 0]
    rows = row0 + i * bn + lax.broadcasted_iota(jnp.int32, dv.shape, 0)
    dv = jnp.where(rows < N, dv, 0.0)
    dv_out[...] = jnp.broadcast_to(dv[:, None], (bn, DEG))


def _k4(h, w1, b1, w2, b2, dw1, db1, dw2, db2, row0=0):
    bn = 512
    nrows = h.shape[0]
    wspec = pl.BlockSpec((D, D), lambda i: (0, 0))
    bspec = pl.BlockSpec((1, D), lambda i: (0, 0))
    sspec = pl.BlockSpec((1, 1), lambda i: (0, 0))
    return pl.pallas_call(
        functools.partial(_k4_body, row0=row0),
        grid=(nrows // bn,),
        in_specs=[
            pl.BlockSpec((bn, DEG, D), lambda i: (i, 0, 0)),
            wspec, bspec, bspec, sspec, wspec, bspec, bspec, sspec,
        ],
        out_specs=[
            pl.BlockSpec((bn, DEG), lambda i: (i, 0)),
            pl.BlockSpec((bn, DEG), lambda i: (i, 0)),
        ],
        out_shape=[
            jax.ShapeDtypeStruct((nrows, DEG), F32),
            jax.ShapeDtypeStruct((nrows, DEG), F32),
        ],
    )(h, w1, b1, w2, b2, dw1, db1, dw2, db2)


def _row_gather(table, idx):
    mesh = plsc.VectorSubcoreMesh(core_axis_name="c", subcore_axis_name="s")
    win = 128
    w = table.shape[1]

    @functools.partial(
        pl.kernel,
        out_type=jax.ShapeDtypeStruct((E, w), table.dtype),
        mesh=mesh,
        compiler_params=_sc_params(tc_tiling=False if w < 128 else None),
    )
    def gk(tab_hbm, idx_hbm, out_hbm):
        def body(i_vmem, o_vmem):
            pltpu.sync_copy(tab_hbm.at[i_vmem.at[0]], o_vmem)

        pltpu.emit_pipeline(
            body,
            grid=(E // win,),
            in_specs=[pl.BlockSpec((1, win), index_map=lambda i: (0, i))],
            out_specs=[pl.BlockSpec((win, w), index_map=lambda i: (i, 0))],
            core_axis_name=("c", "s"),
            dimension_semantics=(pltpu.PARALLEL,),
        )(idx_hbm, out_hbm)

    return gk(table, idx.reshape(1, E))


def _row_gather2(table, idx):
    mesh = plsc.VectorSubcoreMesh(core_axis_name="c", subcore_axis_name="s")
    win = 64
    w = table.shape[1]
    ne = idx.shape[0]
    e_per = ne // 32
    nw = e_per // win
    assert e_per % win == 0
    nbuf = next(d for d in (8, 5, 4, 2, 1) if nw % d == 0)
    ngrp = nw // nbuf

    @functools.partial(
        pl.kernel,
        out_type=jax.ShapeDtypeStruct((ne, w), table.dtype),
        mesh=mesh,
        compiler_params=_sc_params(),
        scratch_types=[
            pltpu.VMEM((e_per,), jnp.int32),
            pltpu.VMEM((nbuf, win, w), table.dtype),
            pltpu.SemaphoreType.DMA,
            pltpu.SemaphoreType.DMA,
        ],
    )
    def gk(tab_hbm, idx_hbm, out_hbm, idx_v, rbuf, gsem, ssem):
        wid = lax.axis_index("s") * 2 + lax.axis_index("c")
        base = wid * e_per
        pltpu.sync_copy(idx_hbm.at[pl.ds(base, e_per)], idx_v)

        def g_copy(wi, b):
            return pltpu.make_async_copy(
                tab_hbm.at[idx_v.at[pl.ds(wi * win, win)]], rbuf.at[b], gsem)

        def s_copy(wi, b):
            return pltpu.make_async_copy(
                rbuf.at[b], out_hbm.at[pl.ds(base + wi * win, win)], ssem)

        for b in range(nbuf):
            g_copy(b, b).start()

        @pl.loop(0, ngrp)
        def _(g):
            for b in range(nbuf):
                wi = g * nbuf + b
                g_copy(wi, b).wait()
                s_copy(wi, b).start()
            for b in range(nbuf):
                wn = (g + 1) * nbuf + b

                @pl.when(wn < nw)
                def _():
                    s_copy(g * nbuf + b, b).wait()
                    g_copy(wn, b).start()

        for b in range(nbuf):
            s_copy((ngrp - 1) * nbuf + b, b).wait()

    return gk(table, idx)


_CHUNK_N = NPAD // 16
_CHUNK_E = _CHUNK_N * DEG
_WIN = 128
_NWIN = _CHUNK_E // _WIN


def _solver(w_flat, dv_flat, ii, rv, adjf, invf, iit, dem_node):
    mesh = plsc.VectorSubcoreMesh(core_axis_name="c", subcore_axis_name="s")
    grp = _CHUNK_N // 16

    @functools.partial(
        pl.kernel,
        out_type=(
            jax.ShapeDtypeStruct((E,), F32),
            jax.ShapeDtypeStruct((16, 16), F32),
            jax.ShapeDtypeStruct((E,), F32),
            jax.ShapeDtypeStruct((E,), F32),
            jax.ShapeDtypeStruct((NPAD,), F32),
        ),
        mesh=mesh,
        compiler_params=_sc_params(),
        scratch_types=[
            pltpu.VMEM((_CHUNK_E,), jnp.int32),
            pltpu.VMEM((_CHUNK_E,), jnp.int32),
            pltpu.VMEM((_CHUNK_E,), jnp.int32),
            pltpu.VMEM((_CHUNK_E,), jnp.int32),
            pltpu.VMEM((_CHUNK_E,), jnp.int32),
            pltpu.VMEM((_CHUNK_E,), F32),
            pltpu.VMEM((_CHUNK_E,), F32),
            pltpu.VMEM((_CHUNK_E,), F32),
            pltpu.VMEM((NPAD,), F32),
            pltpu.VMEM((_CHUNK_N,), F32),
            pltpu.VMEM((_CHUNK_N,), F32),
            pltpu.VMEM((16,), F32),
            pltpu.SemaphoreType.DMA,
        ],
    )
    def sk(w_hbm, dv_hbm, ii_hbm, rv_hbm, adj_hbm, inv_hbm, iit_hbm, dem_hbm,
           dtr_hbm, fc_hbm, wt_hbm, nrm_hbm, s_hbm,
           ii_v, rv_v, adj_v, inv_v, ti_v, gath_v, nrm_v, nin_v,
           tab_v, sv_v, dem_v, acc_v, sem):
        cid = lax.axis_index("c")
        sid = lax.axis_index("s")

        @pl.when(cid == 0)
        def _():
            be = sid * _CHUNK_E
            bn0 = sid * _CHUNK_N
            pltpu.sync_copy(ii_hbm.at[pl.ds(be, _CHUNK_E)], ii_v)
            pltpu.sync_copy(rv_hbm.at[pl.ds(be, _CHUNK_E)], rv_v)
            pltpu.sync_copy(adj_hbm.at[pl.ds(be, _CHUNK_E)], adj_v)
            pltpu.sync_copy(inv_hbm.at[pl.ds(be, _CHUNK_E)], inv_v)
            pltpu.sync_copy(iit_hbm.at[pl.ds(be, _CHUNK_E)], ti_v)
            pltpu.sync_copy(dem_hbm.at[pl.ds(bn0, _CHUNK_N)], dem_v)

            def gather_all(tab, idx_ref, out_ref):
                @pl.loop(0, _NWIN)
                def _start(wi):
                    pltpu.make_async_copy(
                        tab.at[idx_ref.at[pl.ds(wi * _WIN, _WIN)]],
                        out_ref.at[pl.ds(wi * _WIN, _WIN)], sem).start()

                @pl.loop(0, _NWIN)
                def _drain(wi):
                    pltpu.make_async_copy(
                        tab.at[idx_ref.at[pl.ds(wi * _WIN, _WIN)]],
                        out_ref.at[pl.ds(wi * _WIN, _WIN)], sem).wait()

            gather_all(w_hbm, ii_v, gath_v)

            @pl.loop(0, _CHUNK_N)
            def _watt(n):
                e0 = n * DEG
                inc = gath_v[pl.ds(e0, DEG)]
                im = jnp.where(inv_v[pl.ds(e0, DEG)] == N, 1.0, 0.0).astype(F32)
                s = inc - BIG * im
                mx = jnp.max(s)
                ex = jnp.exp(s - mx)
                att = ex / jnp.sum(ex)
                nrm_v[pl.ds(e0, DEG)] = att * inc

            pltpu.sync_copy(nrm_v, wt_hbm.at[pl.ds(be, _CHUNK_E)])
            plsc.subcore_barrier()

            gather_all(wt_hbm, rv_v, gath_v)

            @pl.loop(0, _CHUNK_N)
            def _nrm(n):
                e0 = n * DEG
                nw = gath_v[pl.ds(e0, DEG)]
                mk = jnp.where(adj_v[pl.ds(e0, DEG)] == N, 1.0, 0.0).astype(F32)
                s = nw - BIG * mk
                mx = jnp.max(s)
                ex = jnp.exp(s - mx)
                nrm_v[pl.ds(e0, DEG)] = ex / jnp.sum(ex)

            pltpu.sync_copy(nrm_v, nrm_hbm.at[pl.ds(be, _CHUNK_E)])
            pltpu.sync_copy(dem_v, s_hbm.at[pl.ds(bn0, _CHUNK_N)])

            pltpu.sync_copy(dv_hbm, tab_v)

            @pl.loop(0, _CHUNK_N)
            def _dtr(n):
                e0 = n * DEG
                gath_v[pl.ds(e0, DEG)] = plsc.load_gather(
                    tab_v, [adj_v[pl.ds(e0, DEG)]])

            pltpu.sync_copy(gath_v, dtr_hbm.at[pl.ds(be, _CHUNK_E)])
            plsc.subcore_barrier()

            gather_all(nrm_hbm, ti_v, nin_v)

            for _ in range(FLOW_ITERS):
                pltpu.sync_copy(s_hbm, tab_v)

                @pl.loop(0, grp)
                def _grp(g):
                    acc = jnp.zeros((DEG,), F32)
                    for k in range(DEG):
                        off = g * 256 + k * 16
                        jn = lax.shift_right_logical(ti_v[pl.ds(off, 16)], 4)
                        acc = acc + nin_v[pl.ds(off, 16)] * plsc.load_gather(
                            tab_v, [jn])
                    val = jnp.where(bn0 + g * 16 < N, 1.0, 0.0).astype(F32)
                    sv_v[pl.ds(g * 16, 16)] = (
                        (acc + dem_v[pl.ds(g * 16, 16)]) * val)

                pltpu.sync_copy(sv_v, s_hbm.at[pl.ds(bn0, _CHUNK_N)])
                plsc.subcore_barrier()

            acc_v[...] = jnp.zeros((16,), F32)

            @pl.loop(0, _CHUNK_N)
            def _cost(n):
                sval = plsc.load_gather(sv_v, [jnp.full((16,), n, jnp.int32)])
                f = nrm_v[pl.ds(n * DEG, DEG)] * sval
                acc_v[...] = acc_v[...] + f * f

            pltpu.sync_copy(acc_v, fc_hbm.at[sid])

    return sk(w_flat, dv_flat, ii, rv, adjf, invf, iit, dem_node)


def _k6_body(dtr_ref, dv_ref, adj_ref, dem_ref, fc_ref, out_ref):
    i = pl.program_id(0)
    bn = dtr_ref.shape[0]
    mask = jnp.where(adj_ref[...] == N, 1.0, 0.0).astype(F32)
    dv = dv_ref[...]
    diff = dtr_ref[...] - mask * dv
    df = jnp.zeros_like(diff)
    vel = jnp.zeros_like(diff)
    for _ in range(DUAL_ITERS):
        g = 2.0 * df + diff
        vel = DUAL_MOM * vel - DUAL_STEP * g
        df = jnp.maximum(df + vel, 0.0)
    dflows = df * (1.0 - mask)
    rows = i * bn + lax.broadcasted_iota(jnp.int32, diff.shape, 0)
    valid = jnp.where(rows < N, 1.0, 0.0).astype(F32)
    dfc = (dflows * dflows + diff * dflows) * valid
    lane0 = jnp.where(lax.broadcasted_iota(jnp.int32, diff.shape, 1) == 0,
                      1.0, 0.0).astype(F32)
    dual_demand = jnp.sum(dv * dem_ref[...] * lane0)
    contrib = dual_demand - jnp.sum(dfc)

    @pl.when(i == 0)
    def _():
        out_ref[...] = jnp.reshape(contrib + jnp.sum(fc_ref[...]), (1, 1))

    @pl.when(i > 0)
    def _():
        out_ref[...] = out_ref[...] + jnp.reshape(contrib, (1, 1))


def _k6(dtr16, dv16, adjp, dem16, fc):
    bn = 512
    return pl.pallas_call(
        _k6_body,
        grid=(NPAD // bn,),
        in_specs=[
            pl.BlockSpec((bn, DEG), lambda i: (i, 0)),
            pl.BlockSpec((bn, DEG), lambda i: (i, 0)),
            pl.BlockSpec((bn, DEG), lambda i: (i, 0)),
            pl.BlockSpec((bn, DEG), lambda i: (i, 0)),
            pl.BlockSpec((16, 16), lambda i: (0, 0)),
        ],
        out_specs=pl.BlockSpec((1, 1), lambda i: (0, 0)),
        out_shape=jax.ShapeDtypeStruct((1, 1), F32),
    )(dtr16, dv16, adjp, dem16, fc)


def kernel(demands, node_features, adj_lst, inv_adj_lst, in_indices,
           rev_indices, num_nodes, node_embedding_var, enc_W, enc_b,
           gat_W, gat_b, gat_a,
           gru_Wz, gru_Uz, gru_bz, gru_Wr, gru_Ur, gru_br,
           gru_Wh, gru_Uh, gru_bh,
           dec_W1, dec_b1, dec_W2, dec_b2,
           dual_W1, dual_b1, dual_W2, dual_b2):
    embp = jnp.pad(node_embedding_var, ((0, NPAD - N), (0, 0)))
    featp = jnp.pad(node_features[0], ((0, NPAD - N), (0, 0)))
    adjp = jnp.pad(adj_lst[0], ((0, NPAD - N), (0, 0)), constant_values=N)
    invp = jnp.pad(inv_adj_lst[0], ((0, NPAD - N), (0, 0)), constant_values=N)
    ii = jnp.pad(in_indices[0].reshape(-1), (0, E - N * DEG))
    rv = jnp.pad(rev_indices[0].reshape(-1), (0, E - N * DEG))
    adjf = adjp.reshape(-1)
    invf = invp.reshape(-1)
    dem = jnp.pad(demands[0, :, 0], (0, NPAD - N))
    b2 = lambda v: v.reshape(1, -1)

    e0p = _k1(embp, featp, enc_W[:D], enc_W[D:], b2(enc_b))

    P = 2
    NH = NPAD // P
    EH = E // P
    adjf_p = [adjf[i * EH:(i + 1) * EH] for i in range(P)]
    adjp_p = [adjp[i * NH:(i + 1) * NH] for i in range(P)]
    e0_p = [e0p[i * NH:(i + 1) * NH] for i in range(P)]
    hs = [_row_gather2(e0p, adjf_p[i]).reshape(NH, DEG, D) for i in range(P)]
    for _ in range(2):
        ts = [_k2(hs[i], e0_p[i], gat_W[:D], gat_W[D:], b2(gat_b),
                  row0=i * NH) for i in range(P)]
        t = jnp.concatenate(ts, axis=0)
        gts = [_row_gather2(t, adjf_p[i]).reshape(NH, DEG, D)
               for i in range(P)]
        hs = [_k3(gts[i], hs[i], adjp_p[i], b2(gat_a),
                  gru_Wz, gru_Uz, b2(gru_bz),
                  gru_Wr, gru_Ur, b2(gru_br),
                  gru_Wh, gru_Uh, b2(gru_bh)) for i in range(P)]

    heads = [_k4(hs[i], dec_W1, b2(dec_b1), b2(dec_W2[:, 0]),
                 dec_b2.reshape(1, 1),
                 dual_W1, b2(dual_b1), b2(dual_W2[:, 0]),
                 dual_b2.reshape(1, 1), row0=i * NH) for i in range(P)]
    w16 = jnp.concatenate([hd[0] for hd in heads], axis=0)
    dv16 = jnp.concatenate([hd[1] for hd in heads], axis=0)

    dem16 = jnp.broadcast_to(dem[:, None], (NPAD, DEG))
    iit = ii.reshape(NPAD // 16, 16, 16).swapaxes(1, 2).reshape(-1)
    dtr, fc, _, _, _ = _solver(w16.reshape(-1), dv16[:, 0], ii, rv,
                               adjf, invf, iit, dem)
    loss = _k6(dtr.reshape(NPAD, DEG), dv16, adjp, dem16, fc)
    return loss.reshape(1)
